# Initial kernel scaffold; baseline (speedup 1.0000x reference)
import jax
import jax.numpy as jnp
from jax.experimental import pallas as pl


def kernel(ct, W1, b1, W2, b2, W3, b3, Q):
    raise NotImplementedError("write your pallas kernel here")


# speedup vs baseline: 1.9317x; 1.9317x over previous
import jax
import jax.numpy as jnp
from jax.experimental import pallas as pl
from jax.experimental.pallas import tpu as pltpu

D = 32768
M = 8
H = 64
K = 327
GAMMA = 0.105
BD = 1024
NB = D // BD


def _h2_kernel(ct_ref, W1_ref, b1_ref, W2_ref, b2_ref, h2_ref, acc_ref):
    i = pl.program_id(0)

    @pl.when(i == 0)
    def _init():
        acc_ref[...] = jnp.zeros_like(acc_ref)

    ct_blk = ct_ref[...]
    parts = [
        jnp.dot(ct_blk, W1_ref[m], preferred_element_type=jnp.float32)
        for m in range(M)
    ]
    acc_ref[...] += jnp.concatenate(parts, axis=0)

    @pl.when(i == NB - 1)
    def _finish():
        h1 = jnp.maximum(acc_ref[...] + b1_ref[...], 0.0)
        h2s = [
            jnp.dot(h1[m : m + 1], W2_ref[m], preferred_element_type=jnp.float32)
            for m in range(M)
        ]
        h2_ref[...] = jnp.maximum(jnp.concatenate(h2s, axis=0) + b2_ref[...], 0.0)


def _write_kernel(h2_ref, W3_ref, b3_ref, Q_ref, ct_ref, out_ref, wr_ref, ax_ref):
    i = pl.program_id(0)
    ct_blk = ct_ref[0, pl.ds(i * BD, BD)].reshape(1, BD)
    h2 = h2_ref[...]
    vms = [
        jnp.dot(h2[m : m + 1], W3_ref[m], preferred_element_type=jnp.float32)
        for m in range(M)
    ]
    vm = jnp.concatenate(vms, axis=0) + b3_ref[...]
    gate = jax.nn.sigmoid(Q_ref[...] * ct_blk)
    w = gate * vm
    wr_ref[:, pl.ds(i * BD, BD)] = w
    ax_ref[:, pl.ds(i * BD, BD)] = jax.lax.bitcast_convert_type(w, jnp.int32) & jnp.int32(
        0x7FFFFFFF
    )

    @pl.when(i == NB - 1)
    def _finish():
        ax = ax_ref[...]

        def body(_, lohi):
            lo, hi = lohi
            mid = lo + ((hi - lo + 1) >> 1)
            cnt = jnp.sum((ax >= mid).astype(jnp.int32), axis=1, keepdims=True)
            ge = cnt >= K
            return jnp.where(ge, mid, lo), jnp.where(ge, hi, mid - 1)

        lo0 = jnp.zeros((M, 1), jnp.int32)
        hi0 = jnp.full((M, 1), 0x7F800000, jnp.int32)
        thr, _ = jax.lax.fori_loop(0, 31, body, (lo0, hi0))
        keep = ax >= thr
        ssum = jnp.sum(jnp.where(keep, wr_ref[...], 0.0), axis=0)
        out_ref[0, :] = (1.0 - GAMMA) * ct_ref[0, :] + ssum


def kernel(ct, W1, b1, W2, b2, W3, b3, Q):
    ct2 = ct.reshape(1, D)

    h2 = pl.pallas_call(
        _h2_kernel,
        grid=(NB,),
        in_specs=[
            pl.BlockSpec((1, BD), lambda i: (0, i)),
            pl.BlockSpec((M, BD, H), lambda i: (0, i, 0)),
            pl.BlockSpec((M, H), lambda i: (0, 0)),
            pl.BlockSpec((M, H, H), lambda i: (0, 0, 0)),
            pl.BlockSpec((M, H), lambda i: (0, 0)),
        ],
        out_specs=pl.BlockSpec((M, H), lambda i: (0, 0)),
        out_shape=jax.ShapeDtypeStruct((M, H), jnp.float32),
        scratch_shapes=[pltpu.VMEM((M, H), jnp.float32)],
        compiler_params=pltpu.CompilerParams(
            dimension_semantics=("arbitrary",),
        ),
    )(ct2, W1, b1, W2, b2)

    ct_next = pl.pallas_call(
        _write_kernel,
        grid=(NB,),
        in_specs=[
            pl.BlockSpec((M, H), lambda i: (0, 0)),
            pl.BlockSpec((M, H, BD), lambda i: (0, 0, i)),
            pl.BlockSpec((M, BD), lambda i: (0, i)),
            pl.BlockSpec((M, BD), lambda i: (0, i)),
            pl.BlockSpec((1, D), lambda i: (0, 0)),
        ],
        out_specs=pl.BlockSpec((1, D), lambda i: (0, 0)),
        out_shape=jax.ShapeDtypeStruct((1, D), jnp.float32),
        scratch_shapes=[
            pltpu.VMEM((M, D), jnp.float32),
            pltpu.VMEM((M, D), jnp.int32),
        ],
        compiler_params=pltpu.CompilerParams(
            dimension_semantics=("arbitrary",),
        ),
    )(h2, W3, b3, Q, ct2)

    return ct_next.reshape(D)


# speedup vs baseline: 2.0689x; 1.0710x over previous
import jax
import jax.numpy as jnp
from jax.experimental import pallas as pl
from jax.experimental.pallas import tpu as pltpu

D = 32768
M = 8
H = 64
K = 327
GAMMA = 0.105
BD = 2048
NB = D // BD


def _h2_kernel(ct_ref, W1_ref, b1_ref, W2_ref, b2_ref, h2_ref, acc_ref):
    i = pl.program_id(0)

    @pl.when(i == 0)
    def _init():
        acc_ref[...] = jnp.zeros_like(acc_ref)

    ct_blk = ct_ref[...]
    parts = [
        jnp.dot(ct_blk, W1_ref[m], preferred_element_type=jnp.float32)
        for m in range(M)
    ]
    acc_ref[...] += jnp.concatenate(parts, axis=0)

    @pl.when(i == NB - 1)
    def _finish():
        h1 = jnp.maximum(acc_ref[...] + b1_ref[...], 0.0)
        h2s = [
            jnp.dot(h1[m : m + 1], W2_ref[m], preferred_element_type=jnp.float32)
            for m in range(M)
        ]
        h2_ref[...] = jnp.maximum(jnp.concatenate(h2s, axis=0) + b2_ref[...], 0.0)


def _write_kernel(h2_ref, W3_ref, b3_ref, Q_ref, ct_ref, out_ref, wr_ref, ax_ref):
    i = pl.program_id(0)
    ct_blk = ct_ref[0, pl.ds(i * BD, BD)].reshape(1, BD)
    h2 = h2_ref[...]
    vms = [
        jnp.dot(h2[m : m + 1], W3_ref[m], preferred_element_type=jnp.float32)
        for m in range(M)
    ]
    vm = jnp.concatenate(vms, axis=0) + b3_ref[...]
    gate = jax.nn.sigmoid(Q_ref[...] * ct_blk)
    w = gate * vm
    wr_ref[:, pl.ds(i * BD, BD)] = w
    ax_ref[:, pl.ds(i * BD, BD)] = jax.lax.bitcast_convert_type(w, jnp.int32) & jnp.int32(
        0x7FFFFFFF
    )

    @pl.when(i == NB - 1)
    def _finish():
        ax = ax_ref[...]

        def body(_, lohi):
            lo, hi = lohi
            mid = lo + ((hi - lo + 1) >> 1)
            cnt = jnp.sum((ax >= mid).astype(jnp.int32), axis=1, keepdims=True)
            ge = cnt >= K
            return jnp.where(ge, mid, lo), jnp.where(ge, hi, mid - 1)

        lo0 = jnp.zeros((M, 1), jnp.int32)
        hi0 = jnp.full((M, 1), 0x7F800000, jnp.int32)
        thr, _ = jax.lax.fori_loop(0, 31, body, (lo0, hi0))
        keep = ax >= thr
        ssum = jnp.sum(jnp.where(keep, wr_ref[...], 0.0), axis=0)
        out_ref[0, :] = (1.0 - GAMMA) * ct_ref[0, :] + ssum


def kernel(ct, W1, b1, W2, b2, W3, b3, Q):
    ct2 = ct.reshape(1, D)

    h2 = pl.pallas_call(
        _h2_kernel,
        grid=(NB,),
        in_specs=[
            pl.BlockSpec((1, BD), lambda i: (0, i)),
            pl.BlockSpec((M, BD, H), lambda i: (0, i, 0)),
            pl.BlockSpec((M, H), lambda i: (0, 0)),
            pl.BlockSpec((M, H, H), lambda i: (0, 0, 0)),
            pl.BlockSpec((M, H), lambda i: (0, 0)),
        ],
        out_specs=pl.BlockSpec((M, H), lambda i: (0, 0)),
        out_shape=jax.ShapeDtypeStruct((M, H), jnp.float32),
        scratch_shapes=[pltpu.VMEM((M, H), jnp.float32)],
        compiler_params=pltpu.CompilerParams(
            dimension_semantics=("arbitrary",),
        ),
    )(ct2, W1, b1, W2, b2)

    ct_next = pl.pallas_call(
        _write_kernel,
        grid=(NB,),
        in_specs=[
            pl.BlockSpec((M, H), lambda i: (0, 0)),
            pl.BlockSpec((M, H, BD), lambda i: (0, 0, i)),
            pl.BlockSpec((M, BD), lambda i: (0, i)),
            pl.BlockSpec((M, BD), lambda i: (0, i)),
            pl.BlockSpec((1, D), lambda i: (0, 0)),
        ],
        out_specs=pl.BlockSpec((1, D), lambda i: (0, 0)),
        out_shape=jax.ShapeDtypeStruct((1, D), jnp.float32),
        scratch_shapes=[
            pltpu.VMEM((M, D), jnp.float32),
            pltpu.VMEM((M, D), jnp.int32),
        ],
        compiler_params=pltpu.CompilerParams(
            dimension_semantics=("arbitrary",),
        ),
    )(h2, W3, b3, Q, ct2)

    return ct_next.reshape(D)


# speedup vs baseline: 2.0828x; 1.0067x over previous
import jax
import jax.numpy as jnp
from jax.experimental import pallas as pl
from jax.experimental.pallas import tpu as pltpu

D = 32768
M = 8
H = 64
K = 327
GAMMA = 0.105
BD = 4096
NB = D // BD


def _h2_kernel(ct_ref, W1_ref, b1_ref, W2_ref, b2_ref, h2_ref, acc_ref):
    i = pl.program_id(0)

    @pl.when(i == 0)
    def _init():
        acc_ref[...] = jnp.zeros_like(acc_ref)

    ct_blk = ct_ref[...]
    parts = [
        jnp.dot(ct_blk, W1_ref[m], preferred_element_type=jnp.float32)
        for m in range(M)
    ]
    acc_ref[...] += jnp.concatenate(parts, axis=0)

    @pl.when(i == NB - 1)
    def _finish():
        h1 = jnp.maximum(acc_ref[...] + b1_ref[...], 0.0)
        h2s = [
            jnp.dot(h1[m : m + 1], W2_ref[m], preferred_element_type=jnp.float32)
            for m in range(M)
        ]
        h2_ref[...] = jnp.maximum(jnp.concatenate(h2s, axis=0) + b2_ref[...], 0.0)


def _write_kernel(h2_ref, W3_ref, b3_ref, Q_ref, ct_ref, out_ref, wr_ref, ax_ref):
    i = pl.program_id(0)
    ct_blk = ct_ref[0, pl.ds(i * BD, BD)].reshape(1, BD)
    h2 = h2_ref[...]
    vms = [
        jnp.dot(h2[m : m + 1], W3_ref[m], preferred_element_type=jnp.float32)
        for m in range(M)
    ]
    vm = jnp.concatenate(vms, axis=0) + b3_ref[...]
    gate = jax.nn.sigmoid(Q_ref[...] * ct_blk)
    w = gate * vm
    wr_ref[:, pl.ds(i * BD, BD)] = w
    ax_ref[:, pl.ds(i * BD, BD)] = jax.lax.bitcast_convert_type(w, jnp.int32) & jnp.int32(
        0x7FFFFFFF
    )

    @pl.when(i == NB - 1)
    def _finish():
        ax = ax_ref[...]

        def body(_, lohi):
            lo, hi = lohi
            mid = lo + ((hi - lo + 1) >> 1)
            cnt = jnp.sum((ax >= mid).astype(jnp.int32), axis=1, keepdims=True)
            ge = cnt >= K
            return jnp.where(ge, mid, lo), jnp.where(ge, hi, mid - 1)

        lo0 = jnp.zeros((M, 1), jnp.int32)
        hi0 = jnp.full((M, 1), 0x7F800000, jnp.int32)
        thr, _ = jax.lax.fori_loop(0, 31, body, (lo0, hi0))
        keep = ax >= thr
        ssum = jnp.sum(jnp.where(keep, wr_ref[...], 0.0), axis=0)
        out_ref[0, :] = (1.0 - GAMMA) * ct_ref[0, :] + ssum


def kernel(ct, W1, b1, W2, b2, W3, b3, Q):
    ct2 = ct.reshape(1, D)

    h2 = pl.pallas_call(
        _h2_kernel,
        grid=(NB,),
        in_specs=[
            pl.BlockSpec((1, BD), lambda i: (0, i)),
            pl.BlockSpec((M, BD, H), lambda i: (0, i, 0)),
            pl.BlockSpec((M, H), lambda i: (0, 0)),
            pl.BlockSpec((M, H, H), lambda i: (0, 0, 0)),
            pl.BlockSpec((M, H), lambda i: (0, 0)),
        ],
        out_specs=pl.BlockSpec((M, H), lambda i: (0, 0)),
        out_shape=jax.ShapeDtypeStruct((M, H), jnp.float32),
        scratch_shapes=[pltpu.VMEM((M, H), jnp.float32)],
        compiler_params=pltpu.CompilerParams(
            dimension_semantics=("arbitrary",),
        ),
    )(ct2, W1, b1, W2, b2)

    ct_next = pl.pallas_call(
        _write_kernel,
        grid=(NB,),
        in_specs=[
            pl.BlockSpec((M, H), lambda i: (0, 0)),
            pl.BlockSpec((M, H, BD), lambda i: (0, 0, i)),
            pl.BlockSpec((M, BD), lambda i: (0, i)),
            pl.BlockSpec((M, BD), lambda i: (0, i)),
            pl.BlockSpec((1, D), lambda i: (0, 0)),
        ],
        out_specs=pl.BlockSpec((1, D), lambda i: (0, 0)),
        out_shape=jax.ShapeDtypeStruct((1, D), jnp.float32),
        scratch_shapes=[
            pltpu.VMEM((M, D), jnp.float32),
            pltpu.VMEM((M, D), jnp.int32),
        ],
        compiler_params=pltpu.CompilerParams(
            dimension_semantics=("arbitrary",),
        ),
    )(h2, W3, b3, Q, ct2)

    return ct_next.reshape(D)
